# baseline (speedup 1.0000x reference)
import functools

import jax
import jax.numpy as jnp
import numpy as np
from jax.experimental import pallas as pl
from jax.experimental.pallas import tpu as pltpu
from jax.experimental.pallas import tpu_sc as plsc

_TD = 128
_K = 16
_NB = 4

_F32 = jnp.float32
_BF16 = jnp.bfloat16


def _dot(a, b):
    return jnp.dot(a, b, preferred_element_type=_F32)


def _bdot(a, b):
    return jnp.dot(a.astype(_BF16), b.astype(_BF16), preferred_element_type=_F32)


def _row_bs(blk, w):
    return pl.BlockSpec((blk, w), lambda i: (i, 0))


def _full_bs(shape):
    nd = len(shape)
    return pl.BlockSpec(shape, lambda i: (0,) * nd)


def _cp(ngrid):
    return pltpu.CompilerParams(dimension_semantics=("parallel",) * ngrid)


def _sc_gather(table, idx_flat):
    n_idx = idx_flat.shape[0]
    w = table.shape[1]
    win = 128
    assert n_idx % win == 0, (n_idx, win)
    idx2 = idx_flat.reshape(1, n_idx)
    mesh = plsc.VectorSubcoreMesh(core_axis_name="c", subcore_axis_name="s")

    @functools.partial(
        pl.kernel,
        out_type=jax.ShapeDtypeStruct((n_idx, w), table.dtype),
        mesh=mesh,
    )
    def krn(x_hbm, i_hbm, o_hbm):
        def body(i_vmem, o_vmem):
            pltpu.sync_copy(x_hbm.at[i_vmem.at[0]], o_vmem)

        pltpu.emit_pipeline(
            body,
            grid=(n_idx // win,),
            in_specs=[pl.BlockSpec((1, win), lambda i: (0, i))],
            out_specs=[pl.BlockSpec((win, w), lambda i: (i, 0))],
            core_axis_name=("c", "s"),
            dimension_semantics=(pltpu.PARALLEL,),
        )(i_hbm, o_hbm)

    return krn(table, idx2)


def _embed_krn(x_ref, w0_ref, b0_ref, w1_ref, b1_ref, o_ref):
    h = jnp.maximum(_dot(x_ref[...], w0_ref[...]) + b0_ref[...], 0.0)
    o_ref[...] = jnp.maximum(_dot(h, w1_ref[...]) + b1_ref[...], 0.0)


def _embed(x2, p):
    (w0, b0), (w1, b1) = p
    rows = x2.shape[0]
    blk = 512
    return pl.pallas_call(
        _embed_krn,
        grid=(rows // blk,),
        in_specs=[
            _row_bs(blk, x2.shape[1]),
            _full_bs(w0.shape),
            _full_bs((1, b0.shape[0])),
            _full_bs(w1.shape),
            _full_bs((1, b1.shape[0])),
        ],
        out_specs=_row_bs(blk, w1.shape[1]),
        out_shape=jax.ShapeDtypeStruct((rows, w1.shape[1]), _F32),
        compiler_params=_cp(1),
    )(x2, w0, b0.reshape(1, -1), w1, b1.reshape(1, -1))


_KNN_C = 128
_KNN_NC = 4


def _knn_krn(n_pts, mb, qp_ref, pt_ref, o_ref):
    pid = pl.program_id(0)
    iota = jax.lax.broadcasted_iota(jnp.int32, (_KNN_C, n_pts), 1)
    inf = jnp.float32(jnp.inf)
    for c in range(_KNN_NC):
        r0 = c * _KNN_C
        bidx = (pid * _KNN_NC + c) // mb
        qx = qp_ref[r0 : r0 + _KNN_C, 0:1]
        qy = qp_ref[r0 : r0 + _KNN_C, 1:2]
        qz = qp_ref[r0 : r0 + _KNN_C, 2:3]
        px = pt_ref[bidx, 0:1, :]
        py = pt_ref[bidx, 1:2, :]
        pz = pt_ref[bidx, 2:3, :]
        dx = qx - px
        dy = qy - py
        dz = qz - pz
        d = dx * dx + dy * dy + dz * dz
        off = bidx * n_pts
        for k in range(_K):
            m = jnp.min(d, axis=1, keepdims=True)
            sel = jnp.where(d == m, iota, n_pts)
            am = jnp.min(sel, axis=1, keepdims=True)
            o_ref[r0 : r0 + _KNN_C, k : k + 1] = am + off
            d = jnp.where(iota == am, inf, d)


def _knn(qp, pt):
    bsz, _, n_pts = pt.shape
    rows = qp.shape[0]
    m_rows = rows // bsz
    mb = m_rows // _KNN_C
    blk = _KNN_C * _KNN_NC
    return pl.pallas_call(
        functools.partial(_knn_krn, n_pts, mb),
        grid=(rows // blk,),
        in_specs=[
            pl.BlockSpec((blk, 128), lambda i: (i, 0)),
            pl.BlockSpec((bsz, 3, n_pts), lambda i: (0, 0, 0)),
        ],
        out_specs=pl.BlockSpec((blk, _K), lambda i: (i, 0)),
        out_shape=jax.ShapeDtypeStruct((rows, _K), jnp.int32),
        compiler_params=_cp(1),
    )(qp, pt)


def _pack2(a, b):
    ab = jax.lax.bitcast_convert_type(a.astype(_BF16).astype(_F32), jnp.int32)
    bb = jax.lax.bitcast_convert_type(b.astype(_BF16).astype(_F32), jnp.int32)
    return ab | jax.lax.shift_right_logical(bb, 16)


def _unpack_hi(p):
    return jax.lax.bitcast_convert_type(p & jnp.int32(-65536), _F32)


def _unpack_lo(p):
    return jax.lax.bitcast_convert_type(jax.lax.shift_left(p, 16), _F32)


def _pre_krn(feat_ref, pos_ref, we_ref, be_ref, wq_ref, wk_ref, wv_ref, q_ref, t_ref):
    x = _bdot(feat_ref[...], we_ref[...]) + be_ref[...]
    q_ref[...] = _bdot(x, wq_ref[...])
    kv = _pack2(_bdot(x, wk_ref[...]), _bdot(x, wv_ref[...]))
    t_ref[:, 0:128] = kv
    t_ref[:, 128:256] = jax.lax.bitcast_convert_type(pos_ref[...], jnp.int32)


def _tblock_pre(feat2, ptab, p):
    rows, cin = feat2.shape
    we, be = p["emb"]
    blk = 512
    return pl.pallas_call(
        _pre_krn,
        grid=(rows // blk,),
        in_specs=[
            _row_bs(blk, cin),
            _row_bs(blk, 128),
            _full_bs(we.shape),
            _full_bs((1, _TD)),
            _full_bs((_TD, _TD)),
            _full_bs((_TD, _TD)),
            _full_bs((_TD, _TD)),
        ],
        out_specs=[_row_bs(blk, _TD), _row_bs(blk, 256)],
        out_shape=[
            jax.ShapeDtypeStruct((rows, _TD), _F32),
            jax.ShapeDtypeStruct((rows, 256), jnp.int32),
        ],
        compiler_params=_cp(1),
    )(feat2, ptab, we, be.reshape(1, -1), p["Wq"], p["Wk"], p["Wv"])


def _pe_broadcast(rel3, w3_ref, r_pts, cw):
    acc = rel3[:, :, 0:1] * w3_ref[0:1, :].reshape(1, 1, cw)
    acc = acc + rel3[:, :, 1:2] * w3_ref[1:2, :].reshape(1, 1, cw)
    return acc + rel3[:, :, 2:3] * w3_ref[2:3, :].reshape(1, 1, cw)


def _post_krn(
    q_ref, pq_ref, res_ref, g_ref,
    p0_ref, pb0_ref, p1_ref, pb1_ref,
    a0_ref, ab0_ref, a1_ref, ab1_ref,
    wo_ref, bo_ref, o_ref,
):
    r_pts = q_ref.shape[0]
    g = g_ref[...]
    gk = _unpack_hi(g[:, 0:128])
    gv = _unpack_lo(g[:, 0:128])
    gp3 = jax.lax.bitcast_convert_type(g[:, 128:131], _F32).reshape(r_pts, _K, 3)
    rel3 = gp3 - pq_ref[:, 0:3][:, None, :]
    pe1_3 = _pe_broadcast(rel3, p0_ref, r_pts, _TD) + pb0_ref[...]
    h = jnp.maximum(pe1_3, 0.0).reshape(r_pts * _K, _TD)
    pe = _bdot(h, p1_ref[...]) + pb1_ref[...]
    pe3 = pe.reshape(r_pts, _K, _TD)
    q3 = q_ref[...][:, None, :]
    t3 = q3 - gk.reshape(r_pts, _K, _TD) + pe3
    th = jnp.maximum(_bdot(t3.reshape(r_pts * _K, _TD), a0_ref[...]) + ab0_ref[...], 0.0)
    a = _bdot(th, a1_ref[...]) + ab1_ref[...]
    a3 = a.reshape(r_pts, _K, _TD) * jnp.float32(1.0 / np.sqrt(_TD))
    mx = jnp.max(a3, axis=1, keepdims=True)
    e = jnp.exp(a3 - mx)
    sm = e / jnp.sum(e, axis=1, keepdims=True)
    r = jnp.sum(sm * (gv.reshape(r_pts, _K, _TD) + pe3), axis=1)
    o_ref[...] = _bdot(r, wo_ref[...]) + bo_ref[...] + res_ref[...]


def _tblock_post(q, ptab, res, g, p):
    rows, cin = res.shape
    blk = 128
    (p0, pb0), (p1, pb1) = p["posenc"]
    (a0, ab0), (a1, ab1) = p["attn"]
    wo, bo = p["out"]
    return pl.pallas_call(
        _post_krn,
        grid=(rows // blk,),
        in_specs=[
            _row_bs(blk, _TD),
            _row_bs(blk, 128),
            _row_bs(blk, cin),
            pl.BlockSpec((blk * _K, 256), lambda i: (i, 0)),
            _full_bs((3, _TD)),
            _full_bs((1, 1, _TD)),
            _full_bs((_TD, _TD)),
            _full_bs((1, _TD)),
            _full_bs((_TD, _TD)),
            _full_bs((1, _TD)),
            _full_bs((_TD, _TD)),
            _full_bs((1, _TD)),
            _full_bs((_TD, cin)),
            _full_bs((1, cin)),
        ],
        out_specs=_row_bs(blk, cin),
        out_shape=jax.ShapeDtypeStruct((rows, cin), _F32),
        compiler_params=_cp(1),
    )(
        q, ptab, res, g,
        p0, pb0.reshape(1, 1, -1), p1, pb1.reshape(1, -1),
        a0, ab0.reshape(1, -1), a1, ab1.reshape(1, -1),
        wo, bo.reshape(1, -1),
    )


def _tdpost_krn(fw, np_ref, g_ref, w1_ref, w1p_ref, b1_ref, w2_ref, b2_ref, o_ref):
    r_pts = np_ref.shape[0]
    ch = w2_ref.shape[1]
    g = g_ref[...]
    h0 = _bdot(g[:, 0:fw], w1_ref[...])
    gp3 = g[:, fw : fw + 3].astype(_F32).reshape(r_pts, _K, 3)
    rel3 = gp3 - np_ref[:, 0:3][:, None, :]
    h3 = h0.reshape(r_pts, _K, ch) + _pe_broadcast(rel3, w1p_ref, r_pts, ch) + b1_ref[...]
    h = jnp.maximum(h3, 0.0).reshape(r_pts * _K, ch)
    h2 = jnp.maximum(_bdot(h, w2_ref[...]) + b2_ref[...], 0.0)
    o_ref[...] = jnp.max(h2.reshape(r_pts, _K, ch), axis=1)


def _tdown_post(nptab, g, p, ch_in, fw):
    (w1, b1), (w2, b2) = p
    ch = w2.shape[1]
    rows = nptab.shape[0]
    blk = 128
    w1f = w1[:ch_in]
    w1p = w1[ch_in : ch_in + 3]
    w1pad = jnp.concatenate([w1f, jnp.zeros((fw - ch_in, ch), _F32)], axis=0)
    return pl.pallas_call(
        functools.partial(_tdpost_krn, fw),
        grid=(rows // blk,),
        in_specs=[
            _row_bs(blk, 128),
            pl.BlockSpec((blk * _K, fw + 128), lambda i: (i, 0)),
            _full_bs((fw, ch)),
            _full_bs((3, ch)),
            _full_bs((1, 1, ch)),
            _full_bs((ch, ch)),
            _full_bs((1, ch)),
        ],
        out_specs=_row_bs(blk, ch),
        out_shape=jax.ShapeDtypeStruct((rows, ch), _F32),
        compiler_params=_cp(1),
    )(nptab, g, w1pad, w1p, b1.reshape(1, 1, -1), w2, b2.reshape(1, -1))


def _sub_idx(i, bsz, n_cur):
    idx = jax.random.randint(jax.random.key(100 + i), (bsz, n_cur // 2), 0, n_cur)
    off = jnp.arange(bsz, dtype=idx.dtype)[:, None] * n_cur
    return (idx + off).reshape(-1).astype(jnp.int32)


def kernel(x, params):
    bsz, n0, _ = x.shape
    pos3 = x.reshape(bsz * n0, 3)
    ptab = jnp.concatenate([pos3, jnp.zeros((bsz * n0, 125), _F32)], axis=1)
    pt = pos3.reshape(bsz, n0, 3).transpose(0, 2, 1)
    feat2 = _embed(pos3, params["embedding"])

    p = params["t1"]
    kidx_t1 = _knn(ptab, pt)
    q, ttab = _tblock_pre(feat2, ptab, p)
    nptab = _sc_gather(ptab, _sub_idx(0, bsz, n0))
    g = _sc_gather(ttab, kidx_t1.reshape(-1))
    kidx_td = _knn(nptab, pt)
    feat2 = _tblock_post(q, ptab, feat2, g, p)

    n_cur = n0
    for i in range(_NB):
        m_cur = n_cur // 2
        ch_in = feat2.shape[1]
        fw = max(128, ch_in)
        npt = nptab[:, 0:3].reshape(bsz, m_cur, 3).transpose(0, 2, 1)
        if fw > ch_in:
            featp = jnp.concatenate(
                [feat2, jnp.zeros((feat2.shape[0], fw - ch_in), _F32)], axis=1
            )
        else:
            featp = feat2
        ftab = jnp.concatenate([featp, ptab], axis=1)
        g_td = _sc_gather(ftab, kidx_td.reshape(-1))
        kidx_tf = _knn(nptab, npt)
        feat2 = _tdown_post(nptab, g_td, params["td"][i], ch_in, fw)

        p = params["tf"][i]
        q, ttab = _tblock_pre(feat2, nptab, p)
        if i + 1 < _NB:
            nptab_nx = _sc_gather(nptab, _sub_idx(i + 1, bsz, m_cur))
        g_tf = _sc_gather(ttab, kidx_tf.reshape(-1))
        if i + 1 < _NB:
            kidx_td = _knn(nptab_nx, npt)
        feat2 = _tblock_post(q, nptab, feat2, g_tf, p)

        ptab, pt, n_cur = nptab, npt, m_cur
        if i + 1 < _NB:
            nptab = nptab_nx

    return feat2.reshape(bsz, n_cur, feat2.shape[1])


# speedup vs baseline: 1.1182x; 1.1182x over previous
import functools

import jax
import jax.numpy as jnp
import numpy as np
from jax.experimental import pallas as pl
from jax.experimental.pallas import tpu as pltpu
from jax.experimental.pallas import tpu_sc as plsc

_TD = 128
_K = 16
_NB = 4

_F32 = jnp.float32
_BF16 = jnp.bfloat16


def _dot(a, b):
    return jnp.dot(a, b, preferred_element_type=_F32)


def _bdot(a, b):
    return jnp.dot(a.astype(_BF16), b.astype(_BF16), preferred_element_type=_F32)


def _row_bs(blk, w):
    return pl.BlockSpec((blk, w), lambda i: (i, 0))


def _full_bs(shape):
    nd = len(shape)
    return pl.BlockSpec(shape, lambda i: (0,) * nd)


def _cp(ngrid):
    return pltpu.CompilerParams(dimension_semantics=("parallel",) * ngrid)


def _sc_gather(table, idx_flat):
    n_idx = idx_flat.shape[0]
    w = table.shape[1]
    win = 128
    assert n_idx % win == 0, (n_idx, win)
    idx2 = idx_flat.reshape(1, n_idx)
    mesh = plsc.VectorSubcoreMesh(core_axis_name="c", subcore_axis_name="s")

    @functools.partial(
        pl.kernel,
        out_type=jax.ShapeDtypeStruct((n_idx, w), table.dtype),
        mesh=mesh,
    )
    def krn(x_hbm, i_hbm, o_hbm):
        def body(i_vmem, o_vmem):
            pltpu.sync_copy(x_hbm.at[i_vmem.at[0]], o_vmem)

        pltpu.emit_pipeline(
            body,
            grid=(n_idx // win,),
            in_specs=[pl.BlockSpec((1, win), lambda i: (0, i))],
            out_specs=[pl.BlockSpec((win, w), lambda i: (i, 0))],
            core_axis_name=("c", "s"),
            dimension_semantics=(pltpu.PARALLEL,),
        )(i_hbm, o_hbm)

    return krn(table, idx2)


def _embed_krn(x_ref, w0_ref, b0_ref, w1_ref, b1_ref, o_ref):
    h = jnp.maximum(_dot(x_ref[...], w0_ref[...]) + b0_ref[...], 0.0)
    o_ref[...] = jnp.maximum(_dot(h, w1_ref[...]) + b1_ref[...], 0.0)


def _embed(x2, p):
    (w0, b0), (w1, b1) = p
    rows = x2.shape[0]
    blk = 512
    return pl.pallas_call(
        _embed_krn,
        grid=(rows // blk,),
        in_specs=[
            _row_bs(blk, x2.shape[1]),
            _full_bs(w0.shape),
            _full_bs((1, b0.shape[0])),
            _full_bs(w1.shape),
            _full_bs((1, b1.shape[0])),
        ],
        out_specs=_row_bs(blk, w1.shape[1]),
        out_shape=jax.ShapeDtypeStruct((rows, w1.shape[1]), _F32),
        compiler_params=_cp(1),
    )(x2, w0, b0.reshape(1, -1), w1, b1.reshape(1, -1))


_KNN_C = 128
_KNN_NC = 4


def _knn_krn(n_pts, mb, qp_ref, pt_ref, o_ref):
    pid = pl.program_id(0)
    iota = jax.lax.broadcasted_iota(jnp.int32, (_KNN_C, n_pts), 1).astype(_F32)
    inf = jnp.float32(jnp.inf)
    nbig = jnp.float32(n_pts)
    for c in range(_KNN_NC):
        r0 = c * _KNN_C
        bidx = (pid * _KNN_NC + c) // mb
        qx = qp_ref[r0 : r0 + _KNN_C, 0:1]
        qy = qp_ref[r0 : r0 + _KNN_C, 1:2]
        qz = qp_ref[r0 : r0 + _KNN_C, 2:3]
        px = pt_ref[bidx, 0:1, :]
        py = pt_ref[bidx, 1:2, :]
        pz = pt_ref[bidx, 2:3, :]
        dx = qx - px
        dy = qy - py
        dz = qz - pz
        d = dx * dx + dy * dy + dz * dz
        off = bidx * n_pts
        for k in range(_K):
            m = jnp.min(d, axis=1, keepdims=True)
            sel = jnp.where(d == m, iota, nbig)
            am = jnp.min(sel, axis=1, keepdims=True)
            o_ref[r0 : r0 + _KNN_C, k : k + 1] = am.astype(jnp.int32) + off
            d = jnp.where(iota == am, inf, d)


def _knn(qp, pt):
    bsz, _, n_pts = pt.shape
    rows = qp.shape[0]
    m_rows = rows // bsz
    mb = m_rows // _KNN_C
    blk = _KNN_C * _KNN_NC
    return pl.pallas_call(
        functools.partial(_knn_krn, n_pts, mb),
        grid=(rows // blk,),
        in_specs=[
            pl.BlockSpec((blk, 128), lambda i: (i, 0)),
            pl.BlockSpec((bsz, 3, n_pts), lambda i: (0, 0, 0)),
        ],
        out_specs=pl.BlockSpec((blk, _K), lambda i: (i, 0)),
        out_shape=jax.ShapeDtypeStruct((rows, _K), jnp.int32),
        compiler_params=_cp(1),
    )(qp, pt)


def _pack2(a, b):
    ab = jax.lax.bitcast_convert_type(a.astype(_BF16).astype(_F32), jnp.int32)
    bb = jax.lax.bitcast_convert_type(b.astype(_BF16).astype(_F32), jnp.int32)
    return ab | jax.lax.shift_right_logical(bb, 16)


def _unpack_hi(p):
    return jax.lax.bitcast_convert_type(p & jnp.int32(-65536), _F32)


def _unpack_lo(p):
    return jax.lax.bitcast_convert_type(jax.lax.shift_left(p, 16), _F32)


def _pre_krn(feat_ref, pos_ref, we_ref, be_ref, wq_ref, wk_ref, wv_ref, q_ref, t_ref):
    x = _bdot(feat_ref[...], we_ref[...]) + be_ref[...]
    q_ref[...] = _bdot(x, wq_ref[...])
    kv = _pack2(_bdot(x, wk_ref[...]), _bdot(x, wv_ref[...]))
    t_ref[:, 0:128] = kv
    t_ref[:, 128:256] = jax.lax.bitcast_convert_type(pos_ref[...], jnp.int32)


def _tblock_pre(feat2, ptab, p):
    rows, cin = feat2.shape
    we, be = p["emb"]
    blk = 512
    return pl.pallas_call(
        _pre_krn,
        grid=(rows // blk,),
        in_specs=[
            _row_bs(blk, cin),
            _row_bs(blk, 128),
            _full_bs(we.shape),
            _full_bs((1, _TD)),
            _full_bs((_TD, _TD)),
            _full_bs((_TD, _TD)),
            _full_bs((_TD, _TD)),
        ],
        out_specs=[_row_bs(blk, _TD), _row_bs(blk, 256)],
        out_shape=[
            jax.ShapeDtypeStruct((rows, _TD), _F32),
            jax.ShapeDtypeStruct((rows, 256), jnp.int32),
        ],
        compiler_params=_cp(1),
    )(feat2, ptab, we, be.reshape(1, -1), p["Wq"], p["Wk"], p["Wv"])


def _pe_broadcast(rel3, w3_ref, r_pts, cw):
    acc = rel3[:, :, 0:1] * w3_ref[0:1, :].reshape(1, 1, cw)
    acc = acc + rel3[:, :, 1:2] * w3_ref[1:2, :].reshape(1, 1, cw)
    return acc + rel3[:, :, 2:3] * w3_ref[2:3, :].reshape(1, 1, cw)


def _post_krn(
    q_ref, pq_ref, res_ref, g_ref,
    p0_ref, pb0_ref, p1_ref, pb1_ref,
    a0_ref, ab0_ref, a1_ref, ab1_ref,
    wo_ref, bo_ref, o_ref,
):
    r_pts = q_ref.shape[0]
    g = g_ref[...]
    gk = _unpack_hi(g[:, 0:128])
    gv = _unpack_lo(g[:, 0:128])
    gp3 = jax.lax.bitcast_convert_type(g[:, 128:131], _F32).reshape(r_pts, _K, 3)
    rel3 = gp3 - pq_ref[:, 0:3][:, None, :]
    pe1_3 = _pe_broadcast(rel3, p0_ref, r_pts, _TD) + pb0_ref[...]
    h = jnp.maximum(pe1_3, 0.0).reshape(r_pts * _K, _TD)
    pe = _bdot(h, p1_ref[...]) + pb1_ref[...]
    pe3 = pe.reshape(r_pts, _K, _TD)
    q3 = q_ref[...][:, None, :]
    t3 = q3 - gk.reshape(r_pts, _K, _TD) + pe3
    th = jnp.maximum(_bdot(t3.reshape(r_pts * _K, _TD), a0_ref[...]) + ab0_ref[...], 0.0)
    a = _bdot(th, a1_ref[...]) + ab1_ref[...]
    a3 = a.reshape(r_pts, _K, _TD) * jnp.float32(1.0 / np.sqrt(_TD))
    mx = jnp.max(a3, axis=1, keepdims=True)
    e = jnp.exp(a3 - mx)
    sm = e / jnp.sum(e, axis=1, keepdims=True)
    r = jnp.sum(sm * (gv.reshape(r_pts, _K, _TD) + pe3), axis=1)
    o_ref[...] = _bdot(r, wo_ref[...]) + bo_ref[...] + res_ref[...]


def _tblock_post(q, ptab, res, g, p):
    rows, cin = res.shape
    blk = 128
    (p0, pb0), (p1, pb1) = p["posenc"]
    (a0, ab0), (a1, ab1) = p["attn"]
    wo, bo = p["out"]
    return pl.pallas_call(
        _post_krn,
        grid=(rows // blk,),
        in_specs=[
            _row_bs(blk, _TD),
            _row_bs(blk, 128),
            _row_bs(blk, cin),
            pl.BlockSpec((blk * _K, 256), lambda i: (i, 0)),
            _full_bs((3, _TD)),
            _full_bs((1, 1, _TD)),
            _full_bs((_TD, _TD)),
            _full_bs((1, _TD)),
            _full_bs((_TD, _TD)),
            _full_bs((1, _TD)),
            _full_bs((_TD, _TD)),
            _full_bs((1, _TD)),
            _full_bs((_TD, cin)),
            _full_bs((1, cin)),
        ],
        out_specs=_row_bs(blk, cin),
        out_shape=jax.ShapeDtypeStruct((rows, cin), _F32),
        compiler_params=_cp(1),
    )(
        q, ptab, res, g,
        p0, pb0.reshape(1, 1, -1), p1, pb1.reshape(1, -1),
        a0, ab0.reshape(1, -1), a1, ab1.reshape(1, -1),
        wo, bo.reshape(1, -1),
    )


def _tdpost_krn(fw, np_ref, g_ref, w1_ref, w1p_ref, b1_ref, w2_ref, b2_ref, o_ref):
    r_pts = np_ref.shape[0]
    ch = w2_ref.shape[1]
    g = g_ref[...]
    h0 = _bdot(g[:, 0:fw], w1_ref[...])
    gp3 = g[:, fw : fw + 3].astype(_F32).reshape(r_pts, _K, 3)
    rel3 = gp3 - np_ref[:, 0:3][:, None, :]
    h3 = h0.reshape(r_pts, _K, ch) + _pe_broadcast(rel3, w1p_ref, r_pts, ch) + b1_ref[...]
    h = jnp.maximum(h3, 0.0).reshape(r_pts * _K, ch)
    h2 = jnp.maximum(_bdot(h, w2_ref[...]) + b2_ref[...], 0.0)
    o_ref[...] = jnp.max(h2.reshape(r_pts, _K, ch), axis=1)


def _tdown_post(nptab, g, p, ch_in, fw):
    (w1, b1), (w2, b2) = p
    ch = w2.shape[1]
    rows = nptab.shape[0]
    blk = 128
    w1f = w1[:ch_in]
    w1p = w1[ch_in : ch_in + 3]
    w1pad = jnp.concatenate([w1f, jnp.zeros((fw - ch_in, ch), _F32)], axis=0)
    return pl.pallas_call(
        functools.partial(_tdpost_krn, fw),
        grid=(rows // blk,),
        in_specs=[
            _row_bs(blk, 128),
            pl.BlockSpec((blk * _K, fw + 128), lambda i: (i, 0)),
            _full_bs((fw, ch)),
            _full_bs((3, ch)),
            _full_bs((1, 1, ch)),
            _full_bs((ch, ch)),
            _full_bs((1, ch)),
        ],
        out_specs=_row_bs(blk, ch),
        out_shape=jax.ShapeDtypeStruct((rows, ch), _F32),
        compiler_params=_cp(1),
    )(nptab, g, w1pad, w1p, b1.reshape(1, 1, -1), w2, b2.reshape(1, -1))


def _sub_idx(i, bsz, n_cur):
    idx = jax.random.randint(jax.random.key(100 + i), (bsz, n_cur // 2), 0, n_cur)
    off = jnp.arange(bsz, dtype=idx.dtype)[:, None] * n_cur
    return (idx + off).reshape(-1).astype(jnp.int32)


def kernel(x, params):
    bsz, n0, _ = x.shape
    pos3 = x.reshape(bsz * n0, 3)
    ptab = jnp.concatenate([pos3, jnp.zeros((bsz * n0, 125), _F32)], axis=1)
    pt = pos3.reshape(bsz, n0, 3).transpose(0, 2, 1)
    feat2 = _embed(pos3, params["embedding"])

    p = params["t1"]
    kidx_t1 = _knn(ptab, pt)
    q, ttab = _tblock_pre(feat2, ptab, p)
    nptab = _sc_gather(ptab, _sub_idx(0, bsz, n0))
    g = _sc_gather(ttab, kidx_t1.reshape(-1))
    kidx_td = _knn(nptab, pt)
    feat2 = _tblock_post(q, ptab, feat2, g, p)

    n_cur = n0
    for i in range(_NB):
        m_cur = n_cur // 2
        ch_in = feat2.shape[1]
        fw = max(128, ch_in)
        npt = nptab[:, 0:3].reshape(bsz, m_cur, 3).transpose(0, 2, 1)
        if fw > ch_in:
            featp = jnp.concatenate(
                [feat2, jnp.zeros((feat2.shape[0], fw - ch_in), _F32)], axis=1
            )
        else:
            featp = feat2
        ftab = jnp.concatenate([featp, ptab], axis=1)
        g_td = _sc_gather(ftab, kidx_td.reshape(-1))
        kidx_tf = _knn(nptab, npt)
        feat2 = _tdown_post(nptab, g_td, params["td"][i], ch_in, fw)

        p = params["tf"][i]
        q, ttab = _tblock_pre(feat2, nptab, p)
        if i + 1 < _NB:
            nptab_nx = _sc_gather(nptab, _sub_idx(i + 1, bsz, m_cur))
        g_tf = _sc_gather(ttab, kidx_tf.reshape(-1))
        if i + 1 < _NB:
            kidx_td = _knn(nptab_nx, npt)
        feat2 = _tblock_post(q, nptab, feat2, g_tf, p)

        ptab, pt, n_cur = nptab, npt, m_cur
        if i + 1 < _NB:
            nptab = nptab_nx

    return feat2.reshape(bsz, n_cur, feat2.shape[1])
